# initial kernel scaffold (unmeasured)
import jax
import jax.numpy as jnp
from jax import lax
from jax.experimental import pallas as pl
from jax.experimental.pallas import tpu as pltpu


def kernel(
    x,
):
    def body(*refs):
        pass

    out_shape = jax.ShapeDtypeStruct(..., jnp.float32)
    return pl.pallas_call(body, out_shape=out_shape)(...)


# baseline (device time: 107135 ns/iter reference)
import jax
import jax.numpy as jnp
from jax import lax
from jax.experimental import pallas as pl
from jax.experimental.pallas import tpu as pltpu


def kernel(x):
    m, n = x.shape

    def body(x_ref, out_ref, send_buf, recv_buf, send_sem, recv_sem):
        my_x = lax.axis_index("x")
        my_y = lax.axis_index("y")
        my_z = lax.axis_index("z")
        peer = (1 - my_x, my_y, my_z)

        barrier_sem = pltpu.get_barrier_semaphore()
        pl.semaphore_signal(
            barrier_sem, inc=1, device_id=peer,
            device_id_type=pl.DeviceIdType.MESH,
        )
        pl.semaphore_wait(barrier_sem, 1)

        send_buf[...] = x_ref[...].astype(jnp.bfloat16)
        rdma = pltpu.make_async_remote_copy(
            src_ref=send_buf,
            dst_ref=recv_buf,
            send_sem=send_sem,
            recv_sem=recv_sem,
            device_id=peer,
            device_id_type=pl.DeviceIdType.MESH,
        )
        rdma.start()
        rdma.wait()
        out_ref[...] = x_ref[...] + recv_buf[...].astype(jnp.float32)

    return pl.pallas_call(
        body,
        out_shape=jax.ShapeDtypeStruct((m, n), jnp.float32),
        in_specs=[pl.BlockSpec(memory_space=pltpu.VMEM)],
        out_specs=pl.BlockSpec(memory_space=pltpu.VMEM),
        scratch_shapes=[
            pltpu.VMEM((m, n), jnp.bfloat16),
            pltpu.VMEM((m, n), jnp.bfloat16),
            pltpu.SemaphoreType.DMA,
            pltpu.SemaphoreType.DMA,
        ],
        compiler_params=pltpu.CompilerParams(collective_id=0),
    )(x)


# device time: 77885 ns/iter; 1.3756x vs baseline; 1.3756x over previous
import jax
import jax.numpy as jnp
from jax import lax
from jax.experimental import pallas as pl
from jax.experimental.pallas import tpu as pltpu

NY = 4
NZ = 4
STEPS = 3
S = 256
H = 128


def kernel(x):
    m, n = x.shape
    MESH = pl.DeviceIdType.MESH

    def body(x_ref, out_ref, sraw, rraw, gbuf, xsem_s, xsem_r, ssems, rsems):
        my_x = lax.axis_index("x")
        my_y = lax.axis_index("y")
        my_z = lax.axis_index("z")
        idx = my_y * NZ + my_z

        bar = pltpu.get_barrier_semaphore()

        def sig(dev):
            pl.semaphore_signal(bar, inc=1, device_id=dev, device_id_type=MESH)

        ym = jnp.maximum(my_y - 1, 0)
        yp = jnp.minimum(my_y + 1, NY - 1)
        zm = jnp.maximum(my_z - 1, 0)
        zp = jnp.minimum(my_z + 1, NZ - 1)

        sig((1 - my_x, my_y, my_z))

        @pl.when(my_y > 0)
        def _():
            sig((my_x, ym, my_z))

        @pl.when(my_y < NY - 1)
        def _():
            sig((my_x, yp, my_z))

        @pl.when(my_z > 0)
        def _():
            sig((my_x, my_y, zm))

        @pl.when(my_z < NZ - 1)
        def _():
            sig((my_x, my_y, zp))

        pl.semaphore_wait(bar, 1)

        @pl.when(my_y > 0)
        def _():
            pl.semaphore_wait(bar, 1)

        @pl.when(my_y < NY - 1)
        def _():
            pl.semaphore_wait(bar, 1)

        @pl.when(my_z > 0)
        def _():
            pl.semaphore_wait(bar, 1)

        @pl.when(my_z < NZ - 1)
        def _():
            pl.semaphore_wait(bar, 1)

        row0 = idx * S
        xs = x_ref[pl.ds(row0, S), :]
        sraw[...] = xs.astype(jnp.bfloat16)
        rx = pltpu.make_async_remote_copy(
            src_ref=sraw,
            dst_ref=rraw,
            send_sem=xsem_s,
            recv_sem=xsem_r,
            device_id=(1 - my_x, my_y, my_z),
            device_id_type=MESH,
        )
        rx.start()
        rx.wait()
        ssum = (xs + rraw[...].astype(jnp.float32)).astype(jnp.bfloat16)
        gbuf[pl.ds(my_y, 1), pl.ds(my_z, 1), :, :] = ssum[None, None]

        def reg_p1A(c):
            return gbuf.at[pl.ds(c, 1), pl.ds(my_z, 1), pl.ds(0, H), :]

        def reg_p1B(c):
            return gbuf.at[pl.ds(my_y, 1), pl.ds(c, 1), pl.ds(H, H), :]

        def reg_p2A(c):
            return gbuf.at[:, pl.ds(c, 1), pl.ds(0, H), :]

        def reg_p2B(c):
            return gbuf.at[pl.ds(c, 1), :, pl.ds(H, H), :]

        def dev(axis, d):
            if axis == "y":
                return (my_x, jnp.clip(my_y + d, 0, NY - 1), my_z)
            return (my_x, my_y, jnp.clip(my_z + d, 0, NZ - 1))

        phase1 = [
            (my_y, "y", +1, reg_p1A),
            (my_y, "y", -1, reg_p1A),
            (my_z, "z", +1, reg_p1B),
            (my_z, "z", -1, reg_p1B),
        ]
        phase2 = [
            (my_z, "z", +1, reg_p2A),
            (my_z, "z", -1, reg_p2A),
            (my_y, "y", +1, reg_p2B),
            (my_y, "y", -1, reg_p2B),
        ]

        def send_cond_chunk(pos, d, s):
            if d == +1:
                return (pos < 3) & (pos - s >= 0), jnp.clip(pos - s, 0, 3)
            return (pos > 0) & (pos + s <= 3), jnp.clip(pos + s, 0, 3)

        def recv_cond_chunk(pos, d, s):
            if d == +1:
                return (pos > 0) & (pos - 1 - s >= 0), jnp.clip(pos - 1 - s, 0, 3)
            return (pos < 3) & (pos + 1 + s <= 3), jnp.clip(pos + 1 + s, 0, 3)

        def mk(reg, cc, fidx, s, axis, d):
            return pltpu.make_async_remote_copy(
                src_ref=reg(cc),
                dst_ref=reg(cc),
                send_sem=ssems.at[fidx, s],
                recv_sem=rsems.at[fidx, s],
                device_id=dev(axis, d),
                device_id_type=MESH,
            )

        def run_phase(flows, base):
            for s in range(STEPS):
                for fi, (pos, axis, d, reg) in enumerate(flows):
                    cond, cc = send_cond_chunk(pos, d, s)

                    @pl.when(cond)
                    def _(reg=reg, cc=cc, fidx=base + fi, s=s, axis=axis, d=d):
                        mk(reg, cc, fidx, s, axis, d).start()

                for fi, (pos, axis, d, reg) in enumerate(flows):
                    cond, cc = recv_cond_chunk(pos, d, s)

                    @pl.when(cond)
                    def _(reg=reg, cc=cc, fidx=base + fi, s=s, axis=axis, d=d):
                        mk(reg, cc, fidx, s, axis, -d).wait_recv()

            for s in range(STEPS):
                for fi, (pos, axis, d, reg) in enumerate(flows):
                    cond, cc = send_cond_chunk(pos, d, s)

                    @pl.when(cond)
                    def _(reg=reg, cc=cc, fidx=base + fi, s=s, axis=axis, d=d):
                        mk(reg, cc, fidx, s, axis, d).wait_send()

        run_phase(phase1, 0)
        run_phase(phase2, 4)

        out_ref[...] = gbuf[...].reshape(m, n).astype(jnp.float32)

    return pl.pallas_call(
        body,
        out_shape=jax.ShapeDtypeStruct((m, n), jnp.float32),
        in_specs=[pl.BlockSpec(memory_space=pltpu.VMEM)],
        out_specs=pl.BlockSpec(memory_space=pltpu.VMEM),
        scratch_shapes=[
            pltpu.VMEM((S, n), jnp.bfloat16),
            pltpu.VMEM((S, n), jnp.bfloat16),
            pltpu.VMEM((NY, NZ, S, n), jnp.bfloat16),
            pltpu.SemaphoreType.DMA,
            pltpu.SemaphoreType.DMA,
            pltpu.SemaphoreType.DMA((8, STEPS)),
            pltpu.SemaphoreType.DMA((8, STEPS)),
        ],
        compiler_params=pltpu.CompilerParams(collective_id=0),
    )(x)


# device time: 66040 ns/iter; 1.6223x vs baseline; 1.1794x over previous
import jax
import jax.numpy as jnp
from jax import lax
from jax.experimental import pallas as pl
from jax.experimental.pallas import tpu as pltpu

NY = 4
NZ = 4
STEPS = 3
RX = 1024
S = 192
H = 96


def kernel(x):
    m, n = x.shape
    MESH = pl.DeviceIdType.MESH

    def body(
        x_ref, out_ref, sraw, rraw, sbulk, rbulk, gbuf,
        xsem_s, xsem_r, bsem_s, bsem_r, ssems, rsems,
    ):
        my_x = lax.axis_index("x")
        my_y = lax.axis_index("y")
        my_z = lax.axis_index("z")
        idx = my_y * NZ + my_z

        bar = pltpu.get_barrier_semaphore()

        def sig(dev):
            pl.semaphore_signal(bar, inc=1, device_id=dev, device_id_type=MESH)

        ym = jnp.maximum(my_y - 1, 0)
        yp = jnp.minimum(my_y + 1, NY - 1)
        zm = jnp.maximum(my_z - 1, 0)
        zp = jnp.minimum(my_z + 1, NZ - 1)

        sig((1 - my_x, my_y, my_z))

        @pl.when(my_y > 0)
        def _():
            sig((my_x, ym, my_z))

        @pl.when(my_y < NY - 1)
        def _():
            sig((my_x, yp, my_z))

        @pl.when(my_z > 0)
        def _():
            sig((my_x, my_y, zm))

        @pl.when(my_z < NZ - 1)
        def _():
            sig((my_x, my_y, zp))

        pl.semaphore_wait(bar, 1)

        @pl.when(my_y > 0)
        def _():
            pl.semaphore_wait(bar, 1)

        @pl.when(my_y < NY - 1)
        def _():
            pl.semaphore_wait(bar, 1)

        @pl.when(my_z > 0)
        def _():
            pl.semaphore_wait(bar, 1)

        @pl.when(my_z < NZ - 1)
        def _():
            pl.semaphore_wait(bar, 1)

        xpeer = (1 - my_x, my_y, my_z)
        row0 = RX + idx * S
        xs = x_ref[pl.ds(row0, S), :]
        sraw[...] = xs.astype(jnp.bfloat16)
        rx = pltpu.make_async_remote_copy(
            src_ref=sraw,
            dst_ref=rraw,
            send_sem=xsem_s,
            recv_sem=xsem_r,
            device_id=xpeer,
            device_id_type=MESH,
        )
        rx.start()
        sbulk[...] = x_ref[0:RX, :].astype(jnp.bfloat16)
        rbx = pltpu.make_async_remote_copy(
            src_ref=sbulk,
            dst_ref=rbulk,
            send_sem=bsem_s,
            recv_sem=bsem_r,
            device_id=xpeer,
            device_id_type=MESH,
        )
        rbx.start()
        rx.wait()
        ssum = (xs + rraw[...].astype(jnp.float32)).astype(jnp.bfloat16)
        gbuf[pl.ds(my_y, 1), pl.ds(my_z, 1), :, :] = ssum[None, None]

        def reg_p1A(c):
            return gbuf.at[pl.ds(c, 1), pl.ds(my_z, 1), pl.ds(0, H), :]

        def reg_p1B(c):
            return gbuf.at[pl.ds(my_y, 1), pl.ds(c, 1), pl.ds(H, H), :]

        def reg_p2A(c):
            return gbuf.at[:, pl.ds(c, 1), pl.ds(0, H), :]

        def reg_p2B(c):
            return gbuf.at[pl.ds(c, 1), :, pl.ds(H, H), :]

        def dev(axis, d):
            if axis == "y":
                return (my_x, jnp.clip(my_y + d, 0, NY - 1), my_z)
            return (my_x, my_y, jnp.clip(my_z + d, 0, NZ - 1))

        phase1 = [
            (my_y, "y", +1, reg_p1A),
            (my_y, "y", -1, reg_p1A),
            (my_z, "z", +1, reg_p1B),
            (my_z, "z", -1, reg_p1B),
        ]
        phase2 = [
            (my_z, "z", +1, reg_p2A),
            (my_z, "z", -1, reg_p2A),
            (my_y, "y", +1, reg_p2B),
            (my_y, "y", -1, reg_p2B),
        ]

        def send_cond_chunk(pos, d, s):
            if d == +1:
                return (pos < 3) & (pos - s >= 0), jnp.clip(pos - s, 0, 3)
            return (pos > 0) & (pos + s <= 3), jnp.clip(pos + s, 0, 3)

        def recv_cond_chunk(pos, d, s):
            if d == +1:
                return (pos > 0) & (pos - 1 - s >= 0), jnp.clip(pos - 1 - s, 0, 3)
            return (pos < 3) & (pos + 1 + s <= 3), jnp.clip(pos + 1 + s, 0, 3)

        def mk(reg, cc, fidx, s, axis, d):
            return pltpu.make_async_remote_copy(
                src_ref=reg(cc),
                dst_ref=reg(cc),
                send_sem=ssems.at[fidx, s],
                recv_sem=rsems.at[fidx, s],
                device_id=dev(axis, d),
                device_id_type=MESH,
            )

        def run_phase(flows, base):
            for s in range(STEPS):
                for fi, (pos, axis, d, reg) in enumerate(flows):
                    cond, cc = send_cond_chunk(pos, d, s)

                    @pl.when(cond)
                    def _(reg=reg, cc=cc, fidx=base + fi, s=s, axis=axis, d=d):
                        mk(reg, cc, fidx, s, axis, d).start()

                for fi, (pos, axis, d, reg) in enumerate(flows):
                    cond, cc = recv_cond_chunk(pos, d, s)

                    @pl.when(cond)
                    def _(reg=reg, cc=cc, fidx=base + fi, s=s, axis=axis, d=d):
                        mk(reg, cc, fidx, s, axis, -d).wait_recv()

            for s in range(STEPS):
                for fi, (pos, axis, d, reg) in enumerate(flows):
                    cond, cc = send_cond_chunk(pos, d, s)

                    @pl.when(cond)
                    def _(reg=reg, cc=cc, fidx=base + fi, s=s, axis=axis, d=d):
                        mk(reg, cc, fidx, s, axis, d).wait_send()

        run_phase(phase1, 0)
        run_phase(phase2, 4)

        rbx.wait()
        out_ref[0:RX, :] = x_ref[0:RX, :] + rbulk[...].astype(jnp.float32)
        out_ref[RX:m, :] = gbuf[...].reshape(m - RX, n).astype(jnp.float32)

    return pl.pallas_call(
        body,
        out_shape=jax.ShapeDtypeStruct((m, n), jnp.float32),
        in_specs=[pl.BlockSpec(memory_space=pltpu.VMEM)],
        out_specs=pl.BlockSpec(memory_space=pltpu.VMEM),
        scratch_shapes=[
            pltpu.VMEM((S, n), jnp.bfloat16),
            pltpu.VMEM((S, n), jnp.bfloat16),
            pltpu.VMEM((RX, n), jnp.bfloat16),
            pltpu.VMEM((RX, n), jnp.bfloat16),
            pltpu.VMEM((NY, NZ, S, n), jnp.bfloat16),
            pltpu.SemaphoreType.DMA,
            pltpu.SemaphoreType.DMA,
            pltpu.SemaphoreType.DMA,
            pltpu.SemaphoreType.DMA,
            pltpu.SemaphoreType.DMA((8, STEPS)),
            pltpu.SemaphoreType.DMA((8, STEPS)),
        ],
        compiler_params=pltpu.CompilerParams(collective_id=0),
    )(x)
